# Initial kernel scaffold; baseline (speedup 1.0000x reference)
import jax
import jax.numpy as jnp
from jax.experimental import pallas as pl


def kernel(actions, emb_table, base_action_emb):
    raise NotImplementedError("write your pallas kernel here")


# speedup vs baseline: 3.6070x; 3.6070x over previous
import functools

import jax
import jax.numpy as jnp
from jax import lax
from jax.experimental import pallas as pl
from jax.experimental.pallas import tpu as pltpu
from jax.experimental.pallas import tpu_sc as plsc

_NW = 32
_G = 128
_K = 4


def _fold_fn(table_ref, base_ref, out_ref):
    out_ref[...] = table_ref[...] + base_ref[...]


@functools.lru_cache(maxsize=None)
def _make_gather(BT, D, NG):
    RPW = NG * _G
    mesh = plsc.VectorSubcoreMesh(core_axis_name="c", subcore_axis_name="s")
    nc = mesh.num_cores

    @functools.partial(
        pl.kernel,
        mesh=mesh,
        out_type=jax.ShapeDtypeStruct((BT, D), jnp.float32),
        scratch_types=[
            pltpu.VMEM((NG, _G), jnp.int32),
            pltpu.VMEM((_K, _G, D), jnp.float32),
            pltpu.SemaphoreType.DMA,
            pltpu.SemaphoreType.DMA,
        ],
        compiler_params=pltpu.CompilerParams(use_tc_tiling_on_sc=False),
    )
    def gather(idx_hbm, table_hbm, out_hbm, idx_v, rows_v, gsem, ssem):
        wid = lax.axis_index("s") * nc + lax.axis_index("c")
        pltpu.sync_copy(idx_hbm.at[wid], idx_v)

        def step(i, carry):
            g0 = i * _K
            cps = [
                pltpu.async_copy(table_hbm.at[idx_v.at[g0 + b]], rows_v.at[b], gsem)
                for b in range(_K)
            ]
            for cp in cps:
                cp.wait()
            row0 = wid * RPW + g0 * _G
            scps = [
                pltpu.async_copy(rows_v.at[b], out_hbm.at[pl.ds(row0 + b * _G, _G)], ssem)
                for b in range(_K)
            ]
            for cp in scps:
                cp.wait()
            return carry

        lax.fori_loop(0, NG // _K, step, 0)

    return gather


def kernel(actions, emb_table, base_action_emb):
    B, T = actions.shape
    V, D = emb_table.shape
    BT = B * T
    VP = -(-(V + 1) // 8) * 8

    base = base_action_emb.astype(jnp.float32)
    padded = jnp.zeros((VP, D), jnp.float32).at[:V].set(emb_table.astype(jnp.float32))
    fold = pl.pallas_call(
        _fold_fn,
        out_shape=jax.ShapeDtypeStruct((VP, D), jnp.float32),
    )
    table2 = fold(padded, base.reshape(1, D))

    shifted = jnp.concatenate(
        [jnp.full((B, 1), V, jnp.int32), actions[:, :-1].astype(jnp.int32)], axis=1
    )
    NG = BT // _NW // _G
    idx = shifted.reshape(_NW, NG, _G)

    out = _make_gather(BT, D, NG)(idx, table2)
    return out.reshape(B, T, 1, D)


# speedup vs baseline: 3.6184x; 1.0031x over previous
import functools

import jax
import jax.numpy as jnp
from jax import lax
from jax.experimental import pallas as pl
from jax.experimental.pallas import tpu as pltpu
from jax.experimental.pallas import tpu_sc as plsc

_NW = 32
_G = 128
_K = 4


def _fold_fn(table_ref, base_ref, out_ref):
    out_ref[...] = table_ref[...] + base_ref[...]


@functools.lru_cache(maxsize=None)
def _make_gather(BT, D, NG):
    RPW = NG * _G
    NS = NG // _K
    assert NS % 2 == 0 and NS >= 4
    mesh = plsc.VectorSubcoreMesh(core_axis_name="c", subcore_axis_name="s")
    nc = mesh.num_cores

    @functools.partial(
        pl.kernel,
        mesh=mesh,
        out_type=jax.ShapeDtypeStruct((BT, D), jnp.float32),
        scratch_types=[
            pltpu.VMEM((NG, _G), jnp.int32),
            pltpu.VMEM((2 * _K, _G, D), jnp.float32),
            pltpu.SemaphoreType.DMA,
            pltpu.SemaphoreType.DMA,
            pltpu.SemaphoreType.DMA,
            pltpu.SemaphoreType.DMA,
        ],
        compiler_params=pltpu.CompilerParams(use_tc_tiling_on_sc=False),
    )
    def gather(idx_hbm, table_hbm, out_hbm, idx_v, rows_v, gsem0, gsem1, ssem0, ssem1):
        wid = lax.axis_index("s") * nc + lax.axis_index("c")
        pltpu.sync_copy(idx_hbm.at[wid], idx_v)
        gsem = (gsem0, gsem1)
        ssem = (ssem0, ssem1)

        def fire_g(i, par):
            g0 = i * _K
            for b in range(_K):
                pltpu.async_copy(
                    table_hbm.at[idx_v.at[g0 + b]], rows_v.at[par * _K + b], gsem[par]
                )

        def drain_g(par):
            for b in range(_K):
                pltpu.make_async_copy(
                    table_hbm.at[idx_v.at[0]], rows_v.at[par * _K + b], gsem[par]
                ).wait()

        def fire_s(i, par):
            row0 = wid * RPW + i * _K * _G
            for b in range(_K):
                pltpu.async_copy(
                    rows_v.at[par * _K + b], out_hbm.at[pl.ds(row0 + b * _G, _G)], ssem[par]
                )

        def drain_s(par):
            for b in range(_K):
                pltpu.make_async_copy(
                    rows_v.at[par * _K + b], out_hbm.at[pl.ds(wid * RPW, _G)], ssem[par]
                ).wait()

        fire_g(0, 0)
        drain_g(0)
        fire_s(0, 0)
        fire_g(1, 1)

        def pair(j, carry):
            i1 = 2 * j + 1
            drain_g(1)
            fire_s(i1, 1)
            drain_s(0)
            fire_g(i1 + 1, 0)
            i2 = i1 + 1
            drain_g(0)
            fire_s(i2, 0)
            drain_s(1)
            fire_g(i2 + 1, 1)
            return carry

        lax.fori_loop(0, NS // 2 - 1, pair, 0)
        drain_g(1)
        fire_s(NS - 1, 1)
        drain_s(0)
        drain_s(1)

    return gather


def kernel(actions, emb_table, base_action_emb):
    B, T = actions.shape
    V, D = emb_table.shape
    BT = B * T
    VP = -(-(V + 1) // 8) * 8

    base = base_action_emb.astype(jnp.float32)
    padded = jnp.zeros((VP, D), jnp.float32).at[:V].set(emb_table.astype(jnp.float32))
    fold = pl.pallas_call(
        _fold_fn,
        out_shape=jax.ShapeDtypeStruct((VP, D), jnp.float32),
    )
    table2 = fold(padded, base.reshape(1, D))

    shifted = jnp.concatenate(
        [jnp.full((B, 1), V, jnp.int32), actions[:, :-1].astype(jnp.int32)], axis=1
    )
    NG = BT // _NW // _G
    idx = shifted.reshape(_NW, NG, _G)

    out = _make_gather(BT, D, NG)(idx, table2)
    return out.reshape(B, T, 1, D)


# speedup vs baseline: 6.6930x; 1.8497x over previous
import functools

import jax
import jax.numpy as jnp
from jax import lax
from jax.experimental import pallas as pl
from jax.experimental.pallas import tpu as pltpu
from jax.experimental.pallas import tpu_sc as plsc

_NW = 32


def _fold_fn(table_ref, base_ref, out_ref):
    out_ref[...] = table_ref[...] + base_ref[...]


@functools.lru_cache(maxsize=None)
def _make_gather(B, T, D, VP):
    assert B == _NW * 128 and D % 8 == 0 and T % 2 == 0
    DB = D // 8
    mesh = plsc.VectorSubcoreMesh(core_axis_name="c", subcore_axis_name="s")
    nc = mesh.num_cores

    @functools.partial(
        pl.kernel,
        mesh=mesh,
        out_type=jax.ShapeDtypeStruct((T, DB, _NW, 8, 128), jnp.float32),
        scratch_types=[
            pltpu.VMEM((T, 128), jnp.int32),
            pltpu.VMEM((D * VP,), jnp.float32),
            pltpu.VMEM((2, DB, 8, 128), jnp.float32),
            pltpu.SemaphoreType.DMA,
            pltpu.SemaphoreType.DMA,
        ],
        compiler_params=pltpu.CompilerParams(
            use_tc_tiling_on_sc=False, needs_layout_passes=False
        ),
    )
    def gather(idx_hbm, tab_hbm, out_hbm, idx_v, tab_v, buf_v, ssem0, ssem1):
        wid = lax.axis_index("s") * nc + lax.axis_index("c")
        pltpu.sync_copy(idx_hbm.at[wid], idx_v)
        pltpu.sync_copy(tab_hbm, tab_v)
        ssem = (ssem0, ssem1)

        def chunk(t, par):
            iv = [idx_v[t, pl.ds(16 * k, 16)] for k in range(8)]
            buf = buf_v.at[par]

            def per_dblk(db, carry):
                row0 = db * (8 * VP)
                for di in range(8):
                    base = row0 + di * VP
                    for k in range(8):
                        buf[db, di, pl.ds(16 * k, 16)] = plsc.load_gather(
                            tab_v, [iv[k] + base]
                        )
                return carry

            lax.fori_loop(0, DB, per_dblk, 0)
            pltpu.async_copy(buf, out_hbm.at[t, pl.ds(0, DB), wid], ssem[par])

        def drain(par):
            pltpu.make_async_copy(
                buf_v.at[par], out_hbm.at[0, pl.ds(0, DB), wid], ssem[par]
            ).wait()

        chunk(0, 0)
        chunk(1, 1)

        def pair(i, carry):
            t0 = 2 * i
            drain(0)
            chunk(t0, 0)
            drain(1)
            chunk(t0 + 1, 1)
            return carry

        lax.fori_loop(1, T // 2, pair, 0)
        drain(0)
        drain(1)

    return gather


def kernel(actions, emb_table, base_action_emb):
    B, T = actions.shape
    V, D = emb_table.shape
    VP = -(-(V + 1) // 8) * 8

    base = base_action_emb.astype(jnp.float32)
    padded = jnp.zeros((VP, D), jnp.float32).at[:V].set(emb_table.astype(jnp.float32))
    fold = pl.pallas_call(
        _fold_fn,
        out_shape=jax.ShapeDtypeStruct((VP, D), jnp.float32),
    )
    table2 = fold(padded, base.reshape(1, D))
    tab_t = table2.T.reshape(-1)

    shifted = jnp.concatenate(
        [jnp.full((B, 1), V, jnp.int32), actions[:, :-1].astype(jnp.int32)], axis=1
    )
    idx = shifted.T.reshape(T, _NW, 128).transpose(1, 0, 2)

    out5 = _make_gather(B, T, D, VP)(idx, tab_t)
    return out5.transpose(2, 4, 0, 1, 3).reshape(B, T, 1, D)


# speedup vs baseline: 18.1854x; 2.7171x over previous
import functools

import jax
import jax.numpy as jnp
from jax import lax
from jax.experimental import pallas as pl
from jax.experimental.pallas import tpu as pltpu
from jax.experimental.pallas import tpu_sc as plsc

_NW = 32


def _fold_fn(table_ref, base_ref, out_ref):
    out_ref[...] = table_ref[...] + base_ref[...]


@functools.lru_cache(maxsize=None)
def _make_gather(B, T, D, VP):
    assert B == _NW * 128 and D % 8 == 0 and T % 2 == 0
    DB = D // 8
    mesh = plsc.VectorSubcoreMesh(core_axis_name="c", subcore_axis_name="s")
    nc = mesh.num_cores

    @functools.partial(
        pl.kernel,
        mesh=mesh,
        out_type=jax.ShapeDtypeStruct((T, DB, _NW, 8, 128), jnp.float32),
        scratch_types=[
            pltpu.VMEM((T, 128), jnp.int32),
            pltpu.VMEM((D * VP,), jnp.float32),
            pltpu.VMEM((2, DB, 8, 128), jnp.float32),
            pltpu.SemaphoreType.DMA,
            pltpu.SemaphoreType.DMA,
        ],
        compiler_params=pltpu.CompilerParams(
            use_tc_tiling_on_sc=False, needs_layout_passes=False
        ),
    )
    def gather(idx_hbm, tab_hbm, out_hbm, idx_v, tab_v, buf_v, ssem0, ssem1):
        wid = lax.axis_index("s") * nc + lax.axis_index("c")
        pltpu.sync_copy(idx_hbm.at[wid], idx_v)
        pltpu.sync_copy(tab_hbm, tab_v)
        ssem = (ssem0, ssem1)

        def chunk(t, par):
            iv = [idx_v[t, pl.ds(16 * k, 16)] for k in range(8)]
            buf = buf_v.at[par]

            @plsc.parallel_loop(0, DB, unroll=2)
            def per_dblk(db):
                row0 = db * (8 * VP)
                for di in range(8):
                    base = row0 + di * VP
                    for k in range(8):
                        buf[db, di, pl.ds(16 * k, 16)] = plsc.load_gather(
                            tab_v, [iv[k] + base]
                        )
            pltpu.async_copy(buf, out_hbm.at[t, pl.ds(0, DB), wid], ssem[par])

        def drain(par):
            pltpu.make_async_copy(
                buf_v.at[par], out_hbm.at[0, pl.ds(0, DB), wid], ssem[par]
            ).wait()

        chunk(0, 0)
        chunk(1, 1)

        def pair(i, carry):
            t0 = 2 * i
            drain(0)
            chunk(t0, 0)
            drain(1)
            chunk(t0 + 1, 1)
            return carry

        lax.fori_loop(1, T // 2, pair, 0)
        drain(0)
        drain(1)

    return gather


def kernel(actions, emb_table, base_action_emb):
    B, T = actions.shape
    V, D = emb_table.shape
    VP = -(-(V + 1) // 8) * 8

    base = base_action_emb.astype(jnp.float32)
    padded = jnp.zeros((VP, D), jnp.float32).at[:V].set(emb_table.astype(jnp.float32))
    fold = pl.pallas_call(
        _fold_fn,
        out_shape=jax.ShapeDtypeStruct((VP, D), jnp.float32),
    )
    table2 = fold(padded, base.reshape(1, D))
    tab_t = table2.T.reshape(-1)

    shifted = jnp.concatenate(
        [jnp.full((B, 1), V, jnp.int32), actions[:, :-1].astype(jnp.int32)], axis=1
    )
    idx = shifted.T.reshape(T, _NW, 128).transpose(1, 0, 2)

    out5 = _make_gather(B, T, D, VP)(idx, tab_t)
    return out5.transpose(2, 4, 0, 1, 3).reshape(B, T, 1, D)
